# baseline (speedup 1.0000x reference)
import jax
import jax.numpy as jnp
from jax.experimental import pallas as pl
from jax.experimental.pallas import tpu as pltpu

_BM = 1024
_BN = 1024


def _readout_matmul_kernel(a_ref, w_ref, o_ref, wbf_ref):
    i = pl.program_id(0)
    j = pl.program_id(1)

    @pl.when(i == 0)
    def _cache_weight_tile():
        wbf_ref[pl.ds(j * _BN, _BN), :] = w_ref[...].astype(jnp.bfloat16)

    a = a_ref[...].astype(jnp.bfloat16)
    o_ref[...] = jax.lax.dot_general(
        a, wbf_ref[pl.ds(j * _BN, _BN), :],
        dimension_numbers=(((1,), (1,)), ((), ())),
        preferred_element_type=jnp.float32,
    )


def kernel(embed, emb_weight):
    m, d = embed.shape
    l, _ = emb_weight.shape
    nj = l // _BN
    grid = (m // _BM, nj)
    return pl.pallas_call(
        _readout_matmul_kernel,
        grid=grid,
        in_specs=[
            pl.BlockSpec((_BM, d), lambda i, j: (i, 0)),
            pl.BlockSpec((_BN, d), lambda i, j: (jnp.where(i == 0, j, nj - 1), 0)),
        ],
        out_specs=pl.BlockSpec((_BM, _BN), lambda i, j: (i, j)),
        out_shape=jax.ShapeDtypeStruct((m, l), jnp.float32),
        scratch_shapes=[pltpu.VMEM((l, d), jnp.bfloat16)],
        compiler_params=pltpu.CompilerParams(
            dimension_semantics=("arbitrary", "arbitrary"),
        ),
    )(embed, emb_weight)


# speedup vs baseline: 1.1526x; 1.1526x over previous
import jax
import jax.numpy as jnp
from jax.experimental import pallas as pl
from jax.experimental.pallas import tpu as pltpu

_BM = 1024
_BN = 2048


def _readout_matmul_kernel(a_ref, w_ref, o_ref, wbf_ref):
    i = pl.program_id(0)
    j = pl.program_id(1)

    @pl.when((i == 0) & (j == 0))
    def _cast_weight_once():
        wbf_ref[...] = w_ref[...].astype(jnp.bfloat16)

    a = a_ref[...].astype(jnp.bfloat16)
    o_ref[...] = jax.lax.dot_general(
        a, wbf_ref[pl.ds(j * _BN, _BN), :],
        dimension_numbers=(((1,), (1,)), ((), ())),
        preferred_element_type=jnp.float32,
    )


def kernel(embed, emb_weight):
    m, d = embed.shape
    l, _ = emb_weight.shape
    grid = (m // _BM, l // _BN)
    return pl.pallas_call(
        _readout_matmul_kernel,
        grid=grid,
        in_specs=[
            pl.BlockSpec((_BM, d), lambda i, j: (i, 0)),
            pl.BlockSpec((l, d), lambda i, j: (0, 0)),
        ],
        out_specs=pl.BlockSpec((_BM, _BN), lambda i, j: (i, j)),
        out_shape=jax.ShapeDtypeStruct((m, l), jnp.float32),
        scratch_shapes=[pltpu.VMEM((l, d), jnp.bfloat16)],
        compiler_params=pltpu.CompilerParams(
            dimension_semantics=("arbitrary", "arbitrary"),
        ),
    )(embed, emb_weight)


# speedup vs baseline: 1.1756x; 1.0199x over previous
import jax
import jax.numpy as jnp
from jax.experimental import pallas as pl
from jax.experimental.pallas import tpu as pltpu

_BM = 1024


def _readout_matmul_kernel(a_ref, w_ref, o_ref):
    a = a_ref[...].astype(jnp.bfloat16)
    w = w_ref[...].astype(jnp.bfloat16)
    o_ref[...] = jax.lax.dot_general(
        a, w,
        dimension_numbers=(((1,), (1,)), ((), ())),
        preferred_element_type=jnp.float32,
    )


def kernel(embed, emb_weight):
    m, d = embed.shape
    l, _ = emb_weight.shape
    grid = (m // _BM,)
    return pl.pallas_call(
        _readout_matmul_kernel,
        grid=grid,
        in_specs=[
            pl.BlockSpec((_BM, d), lambda i: (i, 0)),
            pl.BlockSpec((l, d), lambda i: (0, 0)),
        ],
        out_specs=pl.BlockSpec((_BM, l), lambda i: (i, 0)),
        out_shape=jax.ShapeDtypeStruct((m, l), jnp.float32),
    )(embed, emb_weight)
